# Initial kernel scaffold; baseline (speedup 1.0000x reference)
import functools

import jax
import jax.numpy as jnp
from jax import lax
from jax.experimental import pallas as pl
from jax.experimental.pallas import tpu as pltpu
from jax.experimental.pallas import tpu_sc as plsc

B = 16
C = 32
H = 128
W = 128
N = 50000
BN = B * N

NC = 2
NS = 16
NW = NC * NS

CS = 512
GROUPS = CS // 16
CHUNKS = 49
PER_W = CS * CHUNKS
BN_PAD = NW * PER_W

_mesh = plsc.VectorSubcoreMesh(core_axis_name="c", subcore_axis_name="s")


@functools.partial(
    pl.kernel,
    out_type=jax.ShapeDtypeStruct((BN_PAD, C), jnp.float32),
    mesh=_mesh,
    scratch_types=[
        pltpu.VMEM((CS,), jnp.float32),
        pltpu.VMEM((CS,), jnp.float32),
        pltpu.VMEM((CS,), jnp.float32),
        pltpu.VMEM((CS,), jnp.float32),
        pltpu.VMEM((4 * CS,), jnp.int32),
        pltpu.VMEM((4 * CS, C), jnp.float32),
        pltpu.VMEM((CS, C), jnp.float32),
        pltpu.SemaphoreType.DMA,
    ],
)
def _sc_sample(table_hbm, px_hbm, py_hbm, out_hbm,
               px_v, py_v, wx_v, wy_v, idx_v, rows_v, out_v, sem):
    wid = lax.axis_index("s") * NC + lax.axis_index("c")
    base = wid * PER_W
    lane = lax.iota(jnp.int32, 16)

    @pl.loop(0, CHUNKS)
    def _chunk(ci):
        cbase = base + ci * CS
        pltpu.sync_copy(px_hbm.at[pl.ds(cbase, CS)], px_v)
        pltpu.sync_copy(py_hbm.at[pl.ds(cbase, CS)], py_v)

        @pl.loop(0, GROUPS)
        def _indices(g):
            nl = g * 16 + lane
            nf = (cbase + nl).astype(jnp.float32)
            b = jnp.minimum((nf * (1.0 / N)).astype(jnp.int32), B - 1)
            x = jnp.clip(px_v[pl.ds(g * 16, 16)] * (W - 1.0), 0.0, W - 1.0)
            y = jnp.clip(py_v[pl.ds(g * 16, 16)] * (H - 1.0), 0.0, H - 1.0)
            x0 = x.astype(jnp.int32)
            y0 = y.astype(jnp.int32)
            wx_v[pl.ds(g * 16, 16)] = x - x0.astype(jnp.float32)
            wy_v[pl.ds(g * 16, 16)] = y - y0.astype(jnp.float32)
            x1 = jnp.minimum(x0 + 1, W - 1)
            r0 = b * (H * W) + y0 * W
            r1 = r0 + jnp.where(y0 < H - 1, W, 0)
            idx_v[pl.ds(0 * CS + g * 16, 16)] = r0 + x0
            idx_v[pl.ds(1 * CS + g * 16, 16)] = r0 + x1
            idx_v[pl.ds(2 * CS + g * 16, 16)] = r1 + x0
            idx_v[pl.ds(3 * CS + g * 16, 16)] = r1 + x1

        pltpu.async_copy(table_hbm.at[idx_v], rows_v, sem).wait()

        @pl.loop(0, GROUPS)
        def _blend(g):
            nl = g * 16 + lane
            wx = wx_v[pl.ds(g * 16, 16)]
            wy = wy_v[pl.ds(g * 16, 16)]

            @pl.loop(0, C, unroll=4)
            def _chan(c):
                cvec = lane * 0 + c
                v00 = plsc.load_gather(rows_v, [nl, cvec])
                v01 = plsc.load_gather(rows_v, [nl + CS, cvec])
                v10 = plsc.load_gather(rows_v, [nl + 2 * CS, cvec])
                v11 = plsc.load_gather(rows_v, [nl + 3 * CS, cvec])
                top = v00 + wx * (v01 - v00)
                bot = v10 + wx * (v11 - v10)
                plsc.store_scatter(out_v, [nl, cvec], top + wy * (bot - top))

        pltpu.sync_copy(out_v, out_hbm.at[pl.ds(cbase, CS)])


def kernel(grid, positions):
    table = jnp.transpose(grid, (0, 2, 3, 1)).reshape(B * H * W, C)
    px = jnp.pad(positions[..., 0].reshape(BN), (0, BN_PAD - BN))
    py = jnp.pad(positions[..., 1].reshape(BN), (0, BN_PAD - BN))
    out = _sc_sample(table, px, py)
    return out[:BN].reshape(B, N, C)


# speedup vs baseline: 23.6470x; 23.6470x over previous
import functools

import jax
import jax.numpy as jnp
from jax import lax
from jax.experimental import pallas as pl
from jax.experimental.pallas import tpu as pltpu
from jax.experimental.pallas import tpu_sc as plsc

B = 16
C = 32
H = 128
W = 128
N = 50000
BN = B * N

NC = 2
NS = 16
NW = NC * NS

CS = 512
GROUPS = CS // 16
CHUNKS = 49
PER_W = CS * CHUNKS
BN_PAD = NW * PER_W

_mesh = plsc.VectorSubcoreMesh(core_axis_name="c", subcore_axis_name="s")


@functools.partial(
    pl.kernel,
    out_type=jax.ShapeDtypeStruct((BN_PAD * C,), jnp.float32),
    mesh=_mesh,
    scratch_types=[
        pltpu.VMEM((CS,), jnp.float32),
        pltpu.VMEM((CS,), jnp.float32),
        pltpu.VMEM((CS,), jnp.float32),
        pltpu.VMEM((CS,), jnp.float32),
        pltpu.VMEM((4 * CS,), jnp.int32),
        pltpu.VMEM((4 * CS, C), jnp.float32),
        pltpu.VMEM((CS * C,), jnp.float32),
        pltpu.SemaphoreType.DMA,
    ],
    compiler_params=pltpu.CompilerParams(use_tc_tiling_on_sc=False),
)
def _sc_sample(table_hbm, px_hbm, py_hbm, out_hbm,
               px_v, py_v, wx_v, wy_v, idx_v, rows_v, out_v, sem):
    wid = lax.axis_index("s") * NC + lax.axis_index("c")
    base = wid * PER_W
    lane = lax.iota(jnp.int32, 16)

    @pl.loop(0, CHUNKS)
    def _chunk(ci):
        cbase = base + ci * CS
        pltpu.sync_copy(px_hbm.at[pl.ds(cbase, CS)], px_v)
        pltpu.sync_copy(py_hbm.at[pl.ds(cbase, CS)], py_v)

        @pl.loop(0, GROUPS)
        def _indices(g):
            nl = g * 16 + lane
            nf = (cbase + nl).astype(jnp.float32)
            b = jnp.minimum((nf * (1.0 / N)).astype(jnp.int32), B - 1)
            x = jnp.clip(px_v[pl.ds(g * 16, 16)] * (W - 1.0), 0.0, W - 1.0)
            y = jnp.clip(py_v[pl.ds(g * 16, 16)] * (H - 1.0), 0.0, H - 1.0)
            x0 = x.astype(jnp.int32)
            y0 = y.astype(jnp.int32)
            wx_v[pl.ds(g * 16, 16)] = x - x0.astype(jnp.float32)
            wy_v[pl.ds(g * 16, 16)] = y - y0.astype(jnp.float32)
            x1 = jnp.minimum(x0 + 1, W - 1)
            r0 = b * (H * W) + y0 * W
            r1 = r0 + jnp.where(y0 < H - 1, W, 0)
            idx_v[pl.ds(0 * CS + g * 16, 16)] = r0 + x0
            idx_v[pl.ds(1 * CS + g * 16, 16)] = r0 + x1
            idx_v[pl.ds(2 * CS + g * 16, 16)] = r1 + x0
            idx_v[pl.ds(3 * CS + g * 16, 16)] = r1 + x1

        pltpu.async_copy(table_hbm.at[idx_v], rows_v, sem).wait()

        @pl.loop(0, GROUPS)
        def _blend(g):
            wx16 = wx_v[pl.ds(g * 16, 16)]
            wy16 = wy_v[pl.ds(g * 16, 16)]
            for j in range(16):
                nl = g * 16 + j
                wx = wx16[j]
                wy = wy16[j]
                r00 = rows_v.at[nl]
                r01 = rows_v.at[nl + CS]
                r10 = rows_v.at[nl + 2 * CS]
                r11 = rows_v.at[nl + 3 * CS]
                for h in range(C // 16):
                    sl = pl.ds(h * 16, 16)
                    v00 = r00[sl]
                    v01 = r01[sl]
                    v10 = r10[sl]
                    v11 = r11[sl]
                    top = v00 + wx * (v01 - v00)
                    bot = v10 + wx * (v11 - v10)
                    out_v[pl.ds(nl * C + h * 16, 16)] = top + wy * (bot - top)

        pltpu.sync_copy(out_v, out_hbm.at[pl.ds(cbase * C, CS * C)])


def kernel(grid, positions):
    table = jnp.transpose(grid, (0, 2, 3, 1)).reshape(B * H * W, C)
    px = jnp.pad(positions[..., 0].reshape(BN), (0, BN_PAD - BN))
    py = jnp.pad(positions[..., 1].reshape(BN), (0, BN_PAD - BN))
    out = _sc_sample(table, px, py)
    return out[:BN * C].reshape(B, N, C)


# speedup vs baseline: 33.0097x; 1.3959x over previous
import functools

import jax
import jax.numpy as jnp
from jax import lax
from jax.experimental import pallas as pl
from jax.experimental.pallas import tpu as pltpu
from jax.experimental.pallas import tpu_sc as plsc

B = 16
C = 32
H = 128
W = 128
N = 50000
BN = B * N

NC = 2
NS = 16
NW = NC * NS

PER_W = BN // NW
CS = 352
GROUPS = CS // 16
CHUNKS = -(-PER_W // CS)
LAST_BASE = PER_W - CS

_mesh = plsc.VectorSubcoreMesh(core_axis_name="c", subcore_axis_name="s")


@functools.partial(
    pl.kernel,
    out_type=jax.ShapeDtypeStruct((BN * C,), jnp.float32),
    mesh=_mesh,
    scratch_types=[
        [pltpu.VMEM((CS,), jnp.float32) for _ in range(2)],
        [pltpu.VMEM((CS,), jnp.float32) for _ in range(2)],
        [pltpu.VMEM((CS,), jnp.float32) for _ in range(2)],
        [pltpu.VMEM((CS,), jnp.float32) for _ in range(2)],
        [pltpu.VMEM((4 * CS,), jnp.int32) for _ in range(2)],
        [pltpu.VMEM((4 * CS, C), jnp.float32) for _ in range(2)],
        [pltpu.VMEM((CS * C,), jnp.float32) for _ in range(2)],
        [pltpu.SemaphoreType.DMA for _ in range(2)],
        [pltpu.SemaphoreType.DMA for _ in range(2)],
        [pltpu.SemaphoreType.DMA for _ in range(2)],
    ],
    compiler_params=pltpu.CompilerParams(use_tc_tiling_on_sc=False),
)
def _sc_sample(table_hbm, px_hbm, py_hbm, out_hbm,
               px_v, py_v, wx_v, wy_v, idx_v, rows_v, out_v,
               sem_p, sem_g, sem_w):
    wid = lax.axis_index("s") * NC + lax.axis_index("c")
    base = wid * PER_W
    lane = lax.iota(jnp.int32, 16)

    def chunk_base(ci):
        return base + jnp.minimum(ci * CS, LAST_BASE)

    def fire_pos(ci, q):
        cb = chunk_base(ci)
        pltpu.async_copy(px_hbm.at[pl.ds(cb, CS)], px_v[q], sem_p[q])
        pltpu.async_copy(py_hbm.at[pl.ds(cb, CS)], py_v[q], sem_p[q])

    def wait_pos(q):
        pltpu.make_async_copy(px_hbm.at[pl.ds(base, CS)], px_v[q], sem_p[q]).wait()
        pltpu.make_async_copy(py_hbm.at[pl.ds(base, CS)], py_v[q], sem_p[q]).wait()

    def compute_idx(ci, q):
        cb = chunk_base(ci)

        @pl.loop(0, GROUPS)
        def _indices(g):
            nl = g * 16 + lane
            nf = (cb + nl).astype(jnp.float32)
            b = jnp.minimum((nf * (1.0 / N)).astype(jnp.int32), B - 1)
            x = jnp.clip(px_v[q][pl.ds(g * 16, 16)] * (W - 1.0), 0.0, W - 1.0)
            y = jnp.clip(py_v[q][pl.ds(g * 16, 16)] * (H - 1.0), 0.0, H - 1.0)
            x0 = x.astype(jnp.int32)
            y0 = y.astype(jnp.int32)
            wx_v[q][pl.ds(g * 16, 16)] = x - x0.astype(jnp.float32)
            wy_v[q][pl.ds(g * 16, 16)] = y - y0.astype(jnp.float32)
            x1 = jnp.minimum(x0 + 1, W - 1)
            r0 = b * (H * W) + y0 * W
            r1 = r0 + jnp.where(y0 < H - 1, W, 0)
            idx_v[q][pl.ds(0 * CS + g * 16, 16)] = r0 + x0
            idx_v[q][pl.ds(1 * CS + g * 16, 16)] = r0 + x1
            idx_v[q][pl.ds(2 * CS + g * 16, 16)] = r1 + x0
            idx_v[q][pl.ds(3 * CS + g * 16, 16)] = r1 + x1

    def fire_gather(q):
        pltpu.async_copy(table_hbm.at[idx_v[q]], rows_v[q], sem_g[q])

    def wait_gather(q):
        pltpu.make_async_copy(table_hbm.at[idx_v[q]], rows_v[q], sem_g[q]).wait()

    def blend(p):
        @pl.loop(0, GROUPS)
        def _blend(g):
            wx16 = wx_v[p][pl.ds(g * 16, 16)]
            wy16 = wy_v[p][pl.ds(g * 16, 16)]
            for j in range(16):
                nl = g * 16 + j
                wx = wx16[j]
                wy = wy16[j]
                r00 = rows_v[p].at[nl]
                r01 = rows_v[p].at[nl + CS]
                r10 = rows_v[p].at[nl + 2 * CS]
                r11 = rows_v[p].at[nl + 3 * CS]
                for h in range(C // 16):
                    sl = pl.ds(h * 16, 16)
                    v00 = r00[sl]
                    v01 = r01[sl]
                    v10 = r10[sl]
                    v11 = r11[sl]
                    top = v00 + wx * (v01 - v00)
                    bot = v10 + wx * (v11 - v10)
                    out_v[p][pl.ds(nl * C + h * 16, 16)] = top + wy * (bot - top)

    def fire_writeback(ci, p):
        cb = chunk_base(ci)
        pltpu.async_copy(out_v[p], out_hbm.at[pl.ds(cb * C, CS * C)], sem_w[p])

    def wait_writeback(p):
        pltpu.make_async_copy(out_v[p], out_hbm.at[pl.ds(base * C, CS * C)],
                              sem_w[p]).wait()

    pltpu.sync_copy(px_hbm.at[pl.ds(base, CS)], px_v[0])
    pltpu.sync_copy(py_hbm.at[pl.ds(base, CS)], py_v[0])
    compute_idx(0, 0)
    fire_gather(0)
    fire_pos(1, 1)

    @pl.loop(0, CHUNKS, step=2)
    def _chunk(ci):
        for b2 in range(2):
            c = ci + b2
            p = b2
            q = 1 - b2

            @pl.when(c + 1 < CHUNKS)
            def _prep_next():
                wait_pos(q)
                compute_idx(c + 1, q)
                fire_gather(q)

            @pl.when(c + 2 < CHUNKS)
            def _prefetch_pos():
                fire_pos(c + 2, p)

            @pl.when(c >= 2)
            def _drain_wb():
                wait_writeback(p)

            wait_gather(p)
            blend(p)
            fire_writeback(c, p)

    wait_writeback(0)
    wait_writeback(1)


def kernel(grid, positions):
    table = jnp.transpose(grid, (0, 2, 3, 1)).reshape(B * H * W, C)
    px = positions[..., 0].reshape(BN)
    py = positions[..., 1].reshape(BN)
    out = _sc_sample(table, px, py)
    return out.reshape(B, N, C)
